# baseline (speedup 1.0000x reference)
import functools

import jax
import jax.numpy as jnp
from jax import lax
from jax.experimental import pallas as pl
from jax.experimental.pallas import tpu as pltpu
from jax.experimental.pallas import tpu_sc as plsc

N, C_IN, H, W_ = 8, 384, 32, 32
HD, FC, CS, FS = 384, 8, 8, 2
C2 = 2 * HD
SH = H // FS
SW = W_ // FS
L = SH * SW
S = CS * CS
SC2 = C2 // FC
SCH = SC2 // 2
NFOLD = N * FS * FS
M = NFOLD * FC
NW = 32
MPW = M // NW
NLANE = 16


def _sim_kernel(a_ref, wp_ref, bp_ref, ab_ref,
                vals_ref, idx_ref, xv_ref, cv_ref):
    a = a_ref[0]
    alpha = ab_ref[0]
    beta = ab_ref[1]
    xt = jnp.dot(a, wp_ref[...].T, preferred_element_type=jnp.float32,
                 precision=jax.lax.Precision.DEFAULT)
    xt = xt + bp_ref[...]
    r_i = jax.lax.broadcasted_iota(jnp.int32, (S, L), 0)
    l_i = jax.lax.broadcasted_iota(jnp.int32, (S, L), 1)
    sel = ((l_i // SW) // (SH // CS) == r_i // CS) & (
        (l_i % SW) // (SW // CS) == r_i % CS)
    P = jnp.where(sel, 1.0 / ((SH // CS) * (SW // CS)), 0.0)
    ct = jnp.dot(P, xt, preferred_element_type=jnp.float32,
                 precision=jax.lax.Precision.HIGHEST)

    vals_all, idx_all = [], []
    for h in range(FC):
        base = h * SC2
        xp = xt[:, base:base + SCH]
        xv = xt[:, base + SCH:base + SC2]
        cp = ct[:, base:base + SCH]
        cv = ct[:, base + SCH:base + SC2]
        xn = xp / jnp.maximum(
            jnp.sqrt(jnp.sum(xp * xp, axis=1, keepdims=True)), 1e-12)
        cn = cp / jnp.maximum(
            jnp.sqrt(jnp.sum(cp * cp, axis=1, keepdims=True)), 1e-12)
        sim = jnp.dot(xn, cn.T, preferred_element_type=jnp.float32,
                      precision=jax.lax.Precision.DEFAULT)
        sim = jax.nn.sigmoid(alpha * sim + beta)
        vals_all.append(jnp.max(sim, axis=1))
        idx_all.append(jnp.argmax(sim, axis=1).astype(jnp.int32))
        xv_ref[0, h] = xv.T
        cv_ref[0, h] = cv.T
    vals_ref[0] = jnp.stack(vals_all, axis=0)
    idx_ref[0] = jnp.stack(idx_all, axis=0)


def _merge_kernel(d_ref, wm_ref, bm_ref, out_ref):
    out = jax.lax.dot_general(
        d_ref[0], wm_ref[...], (((0,), (1,)), ((), ())),
        preferred_element_type=jnp.float32,
        precision=jax.lax.Precision.HIGHEST)
    out_ref[0] = out + bm_ref[...]


def _route_sc(idx_hbm, vals_hbm, xv_hbm, cv_hbm, out_hbm,
              idx_v, vals_v, xv_v, agg_v, den_v, disp_v):
    wid = lax.axis_index("s") * 2 + lax.axis_index("c")
    pltpu.sync_copy(idx_hbm.at[wid], idx_v)
    pltpu.sync_copy(vals_hbm.at[wid], vals_v)

    def group_body(j, _):
        m = wid * MPW + j
        pltpu.sync_copy(xv_hbm.at[m], xv_v)
        pltpu.sync_copy(cv_hbm.at[m], agg_v)
        for t in range(S // NLANE):
            den_v[pl.ds(t * NLANE, NLANE)] = jnp.full(
                (NLANE,), 1.0, jnp.float32)

        def scat_body(c, _):
            off = j * L + c * NLANE
            iv = idx_v[pl.ds(off, NLANE)]
            wv = vals_v[pl.ds(off, NLANE)]
            plsc.addupdate_scatter(den_v, [iv], wv)
            for col in range(SCH):
                xcol = xv_v[pl.ds(col * L + c * NLANE, NLANE)]
                plsc.addupdate_scatter(agg_v, [iv + col * S], wv * xcol)
            return 0

        lax.fori_loop(0, L // NLANE, scat_body, 0)

        for t in range(S // NLANE):
            rden = 1.0 / den_v[pl.ds(t * NLANE, NLANE)]
            for col in range(SCH):
                off = col * S + t * NLANE
                agg_v[pl.ds(off, NLANE)] = agg_v[pl.ds(off, NLANE)] * rden

        def disp_body(c, _):
            off = j * L + c * NLANE
            iv = idx_v[pl.ds(off, NLANE)]
            wv = vals_v[pl.ds(off, NLANE)]
            for col in range(SCH):
                g = plsc.load_gather(agg_v, [iv + col * S])
                disp_v[pl.ds(col * L + c * NLANE, NLANE)] = wv * g
            return 0

        lax.fori_loop(0, L // NLANE, disp_body, 0)
        pltpu.sync_copy(disp_v, out_hbm.at[m])
        return 0

    lax.fori_loop(0, MPW, group_body, 0)


@functools.partial(jax.jit, static_argnames=("interpret",))
def kernel(x, W_proj, b_proj, W_merge, b_merge, alpha, beta, interpret=False):
    a = x.reshape(N, C_IN, FS, SH, FS, SW).transpose(0, 2, 4, 3, 5, 1)
    a = a.reshape(NFOLD, L, C_IN)
    ab = jnp.concatenate([alpha, beta]).astype(jnp.float32)
    vals, idx, xv_t, cv_t = pl.pallas_call(
        _sim_kernel,
        grid=(NFOLD,),
        in_specs=[
            pl.BlockSpec((1, L, C_IN), lambda f: (f, 0, 0)),
            pl.BlockSpec((C2, C_IN), lambda f: (0, 0)),
            pl.BlockSpec((C2,), lambda f: (0,)),
            pl.BlockSpec(memory_space=pltpu.SMEM),
        ],
        out_specs=[
            pl.BlockSpec((1, FC, L), lambda f: (f, 0, 0)),
            pl.BlockSpec((1, FC, L), lambda f: (f, 0, 0)),
            pl.BlockSpec((1, FC, SCH, L), lambda f: (f, 0, 0, 0)),
            pl.BlockSpec((1, FC, SCH, S), lambda f: (f, 0, 0, 0)),
        ],
        out_shape=[
            jax.ShapeDtypeStruct((NFOLD, FC, L), jnp.float32),
            jax.ShapeDtypeStruct((NFOLD, FC, L), jnp.int32),
            jax.ShapeDtypeStruct((NFOLD, FC, SCH, L), jnp.float32),
            jax.ShapeDtypeStruct((NFOLD, FC, SCH, S), jnp.float32),
        ],
        interpret=interpret,
    )(a, W_proj, b_proj, ab)

    idx_w = idx.reshape(NW, MPW * L)
    vals_w = vals.reshape(NW, MPW * L)
    xv_w = xv_t.reshape(M, SCH * L)
    cv_w = cv_t.reshape(M, SCH * S)

    route = functools.partial(
        pl.kernel,
        mesh=plsc.VectorSubcoreMesh(core_axis_name="c", subcore_axis_name="s"),
        compiler_params=pltpu.CompilerParams(needs_layout_passes=False),
        out_type=jax.ShapeDtypeStruct((M, SCH * L), jnp.float32),
        scratch_types=[
            pltpu.VMEM((MPW * L,), jnp.int32),
            pltpu.VMEM((MPW * L,), jnp.float32),
            pltpu.VMEM((SCH * L,), jnp.float32),
            pltpu.VMEM((SCH * S,), jnp.float32),
            pltpu.VMEM((2 * S,), jnp.float32),
            pltpu.VMEM((SCH * L,), jnp.float32),
        ],
    )(_route_sc)
    disp = route(idx_w, vals_w, xv_w, cv_w)

    disp_cm = disp.reshape(NFOLD, FC * SCH, L)
    out = pl.pallas_call(
        _merge_kernel,
        grid=(NFOLD,),
        in_specs=[
            pl.BlockSpec((1, FC * SCH, L), lambda f: (f, 0, 0)),
            pl.BlockSpec((C_IN, HD), lambda f: (0, 0)),
            pl.BlockSpec((C_IN,), lambda f: (0,)),
        ],
        out_specs=pl.BlockSpec((1, L, C_IN), lambda f: (f, 0, 0)),
        out_shape=jax.ShapeDtypeStruct((NFOLD, L, C_IN), jnp.float32),
        interpret=interpret,
    )(disp_cm, W_merge, b_merge)
    out = out.reshape(N, FS, FS, SH, SW, C_IN).transpose(0, 5, 1, 3, 2, 4)
    return out.reshape(N, C_IN, H, W_)


# speedup vs baseline: 1.0284x; 1.0284x over previous
import functools

import jax
import jax.numpy as jnp
from jax import lax
from jax.experimental import pallas as pl
from jax.experimental.pallas import tpu as pltpu
from jax.experimental.pallas import tpu_sc as plsc

N, C_IN, H, W_ = 8, 384, 32, 32
HD, FC, CS, FS = 384, 8, 8, 2
C2 = 2 * HD
SH = H // FS
SW = W_ // FS
L = SH * SW
S = CS * CS
SC2 = C2 // FC
SCH = SC2 // 2
NFOLD = N * FS * FS
M = NFOLD * FC
NW = 32
MPW = M // NW
NLANE = 16


def _sim_kernel(a_ref, wp_ref, bp_ref, ab_ref,
                vals_ref, idx_ref, xv_ref, cv_ref):
    a = a_ref[0]
    alpha = ab_ref[0]
    beta = ab_ref[1]
    xt = jnp.dot(a, wp_ref[...].T, preferred_element_type=jnp.float32,
                 precision=jax.lax.Precision.DEFAULT)
    xt = xt + bp_ref[...]
    r_i = jax.lax.broadcasted_iota(jnp.int32, (S, L), 0)
    l_i = jax.lax.broadcasted_iota(jnp.int32, (S, L), 1)
    sel = ((l_i // SW) // (SH // CS) == r_i // CS) & (
        (l_i % SW) // (SW // CS) == r_i % CS)
    P = jnp.where(sel, 1.0 / ((SH // CS) * (SW // CS)), 0.0)
    ct = jnp.dot(P, xt, preferred_element_type=jnp.float32,
                 precision=jax.lax.Precision.HIGHEST)

    vals_all, idx_all = [], []
    for h in range(FC):
        base = h * SC2
        xp = xt[:, base:base + SCH]
        xv = xt[:, base + SCH:base + SC2]
        cp = ct[:, base:base + SCH]
        cv = ct[:, base + SCH:base + SC2]
        xn = xp / jnp.maximum(
            jnp.sqrt(jnp.sum(xp * xp, axis=1, keepdims=True)), 1e-12)
        cn = cp / jnp.maximum(
            jnp.sqrt(jnp.sum(cp * cp, axis=1, keepdims=True)), 1e-12)
        sim = jnp.dot(xn, cn.T, preferred_element_type=jnp.float32,
                      precision=jax.lax.Precision.DEFAULT)
        sim = jax.nn.sigmoid(alpha * sim + beta)
        vals = jnp.max(sim, axis=1)
        vals_all.append(vals)
        idx_all.append(jnp.argmax(sim, axis=1).astype(jnp.int32))
        xv_ref[0, h] = (vals[:, None] * xv).T
        cv_ref[0, h] = cv.T
    vals_ref[0] = jnp.stack(vals_all, axis=0)
    idx_ref[0] = jnp.stack(idx_all, axis=0)


def _merge_kernel(d_ref, wm_ref, bm_ref, out_ref):
    out = jax.lax.dot_general(
        d_ref[0], wm_ref[...], (((0,), (1,)), ((), ())),
        preferred_element_type=jnp.float32,
        precision=jax.lax.Precision.HIGHEST)
    out_ref[0] = out + bm_ref[...]


def _route_sc(idx_hbm, vals_hbm, xv_hbm, cv_hbm, out_hbm,
              idx_v, vals_v, xv_v, agg_v, den_v, disp_v):
    wid = lax.axis_index("s") * 2 + lax.axis_index("c")
    pltpu.sync_copy(idx_hbm.at[wid], idx_v)
    pltpu.sync_copy(vals_hbm.at[wid], vals_v)

    def group_body(j, _):
        m = wid * MPW + j
        pltpu.sync_copy(xv_hbm.at[m], xv_v)
        pltpu.sync_copy(cv_hbm.at[m], agg_v)
        for t in range(S // NLANE):
            den_v[pl.ds(t * NLANE, NLANE)] = jnp.full(
                (NLANE,), 1.0, jnp.float32)

        def scat_body(c, _):
            off = j * L + c * NLANE
            iv = idx_v[pl.ds(off, NLANE)]
            wv = vals_v[pl.ds(off, NLANE)]
            plsc.addupdate_scatter(den_v, [iv], wv)
            for col in range(SCH):
                xcol = xv_v[pl.ds(col * L + c * NLANE, NLANE)]
                plsc.addupdate_scatter(agg_v, [iv + col * S], xcol)
            return 0

        lax.fori_loop(0, L // NLANE, scat_body, 0)

        def disp_body(c, _):
            off = j * L + c * NLANE
            iv = idx_v[pl.ds(off, NLANE)]
            wv = vals_v[pl.ds(off, NLANE)]
            wr = wv / plsc.load_gather(den_v, [iv])
            for col in range(SCH):
                g = plsc.load_gather(agg_v, [iv + col * S])
                disp_v[pl.ds(col * L + c * NLANE, NLANE)] = wr * g
            return 0

        lax.fori_loop(0, L // NLANE, disp_body, 0)
        pltpu.sync_copy(disp_v, out_hbm.at[m])
        return 0

    lax.fori_loop(0, MPW, group_body, 0)


@functools.partial(jax.jit, static_argnames=("interpret",))
def kernel(x, W_proj, b_proj, W_merge, b_merge, alpha, beta, interpret=False):
    a = x.reshape(N, C_IN, FS, SH, FS, SW).transpose(0, 2, 4, 3, 5, 1)
    a = a.reshape(NFOLD, L, C_IN)
    ab = jnp.concatenate([alpha, beta]).astype(jnp.float32)
    vals, idx, xv_t, cv_t = pl.pallas_call(
        _sim_kernel,
        grid=(NFOLD,),
        in_specs=[
            pl.BlockSpec((1, L, C_IN), lambda f: (f, 0, 0)),
            pl.BlockSpec((C2, C_IN), lambda f: (0, 0)),
            pl.BlockSpec((C2,), lambda f: (0,)),
            pl.BlockSpec(memory_space=pltpu.SMEM),
        ],
        out_specs=[
            pl.BlockSpec((1, FC, L), lambda f: (f, 0, 0)),
            pl.BlockSpec((1, FC, L), lambda f: (f, 0, 0)),
            pl.BlockSpec((1, FC, SCH, L), lambda f: (f, 0, 0, 0)),
            pl.BlockSpec((1, FC, SCH, S), lambda f: (f, 0, 0, 0)),
        ],
        out_shape=[
            jax.ShapeDtypeStruct((NFOLD, FC, L), jnp.float32),
            jax.ShapeDtypeStruct((NFOLD, FC, L), jnp.int32),
            jax.ShapeDtypeStruct((NFOLD, FC, SCH, L), jnp.float32),
            jax.ShapeDtypeStruct((NFOLD, FC, SCH, S), jnp.float32),
        ],
        interpret=interpret,
    )(a, W_proj, b_proj, ab)

    idx_w = idx.reshape(NW, MPW * L)
    vals_w = vals.reshape(NW, MPW * L)
    xv_w = xv_t.reshape(M, SCH * L)
    cv_w = cv_t.reshape(M, SCH * S)

    route = functools.partial(
        pl.kernel,
        mesh=plsc.VectorSubcoreMesh(core_axis_name="c", subcore_axis_name="s"),
        compiler_params=pltpu.CompilerParams(needs_layout_passes=False),
        out_type=jax.ShapeDtypeStruct((M, SCH * L), jnp.float32),
        scratch_types=[
            pltpu.VMEM((MPW * L,), jnp.int32),
            pltpu.VMEM((MPW * L,), jnp.float32),
            pltpu.VMEM((SCH * L,), jnp.float32),
            pltpu.VMEM((SCH * S,), jnp.float32),
            pltpu.VMEM((2 * S,), jnp.float32),
            pltpu.VMEM((SCH * L,), jnp.float32),
        ],
    )(_route_sc)
    disp = route(idx_w, vals_w, xv_w, cv_w)

    disp_cm = disp.reshape(NFOLD, FC * SCH, L)
    out = pl.pallas_call(
        _merge_kernel,
        grid=(NFOLD,),
        in_specs=[
            pl.BlockSpec((1, FC * SCH, L), lambda f: (f, 0, 0)),
            pl.BlockSpec((C_IN, HD), lambda f: (0, 0)),
            pl.BlockSpec((C_IN,), lambda f: (0,)),
        ],
        out_specs=pl.BlockSpec((1, L, C_IN), lambda f: (f, 0, 0)),
        out_shape=jax.ShapeDtypeStruct((NFOLD, L, C_IN), jnp.float32),
        interpret=interpret,
    )(disp_cm, W_merge, b_merge)
    out = out.reshape(N, FS, FS, SH, SW, C_IN).transpose(0, 5, 1, 3, 2, 4)
    return out.reshape(N, C_IN, H, W_)


# speedup vs baseline: 1.1374x; 1.1060x over previous
import functools

import jax
import jax.numpy as jnp
from jax import lax
from jax.experimental import pallas as pl
from jax.experimental.pallas import tpu as pltpu
from jax.experimental.pallas import tpu_sc as plsc

N, C_IN, H, W_ = 8, 384, 32, 32
HD, FC, CS, FS = 384, 8, 8, 2
C2 = 2 * HD
SH = H // FS
SW = W_ // FS
L = SH * SW
S = CS * CS
SC2 = C2 // FC
SCH = SC2 // 2
NFOLD = N * FS * FS
M = NFOLD * FC
NW = 32
MPW = M // NW
NLANE = 16


def _sim_kernel(a_ref, wp_ref, bp_ref, ab_ref,
                vals_ref, idx_ref, xv_ref, cv_ref):
    a = a_ref[0]
    alpha = ab_ref[0]
    beta = ab_ref[1]
    xt = jnp.dot(a, wp_ref[...].T, preferred_element_type=jnp.float32,
                 precision=jax.lax.Precision.DEFAULT)
    xt = xt + bp_ref[...]
    r_i = jax.lax.broadcasted_iota(jnp.int32, (S, L), 0)
    l_i = jax.lax.broadcasted_iota(jnp.int32, (S, L), 1)
    sel = ((l_i // SW) // (SH // CS) == r_i // CS) & (
        (l_i % SW) // (SW // CS) == r_i % CS)
    P = jnp.where(sel, 1.0 / ((SH // CS) * (SW // CS)), 0.0)
    ct = jnp.dot(P, xt, preferred_element_type=jnp.float32,
                 precision=jax.lax.Precision.HIGHEST)

    vals_all, idx_all = [], []
    for h in range(FC):
        base = h * SC2
        xp = xt[:, base:base + SCH]
        xv = xt[:, base + SCH:base + SC2]
        cp = ct[:, base:base + SCH]
        cv = ct[:, base + SCH:base + SC2]
        xn = xp / jnp.maximum(
            jnp.sqrt(jnp.sum(xp * xp, axis=1, keepdims=True)), 1e-12)
        cn = cp / jnp.maximum(
            jnp.sqrt(jnp.sum(cp * cp, axis=1, keepdims=True)), 1e-12)
        sim = jnp.dot(xn, cn.T, preferred_element_type=jnp.float32,
                      precision=jax.lax.Precision.DEFAULT)
        sim = jax.nn.sigmoid(alpha * sim + beta)
        vals = jnp.max(sim, axis=1)
        vals_all.append(vals)
        idx_all.append(jnp.argmax(sim, axis=1).astype(jnp.int32))
        xv_ref[0, h] = (vals[:, None] * xv).T
        cv_ref[0, h] = cv.T
    vals_ref[0] = jnp.stack(vals_all, axis=0)
    idx_ref[0] = jnp.stack(idx_all, axis=0)


def _merge_kernel(d_ref, wm_ref, bm_ref, out_ref):
    out = jax.lax.dot_general(
        d_ref[0], wm_ref[...], (((0,), (1,)), ((), ())),
        preferred_element_type=jnp.float32,
        precision=jax.lax.Precision.HIGHEST)
    out_ref[0] = out + bm_ref[...]


def _route_sc(mpw, idx_hbm, vals_hbm, xv_hbm, cv_hbm, out_hbm,
              idx_v, vals_v, xv_v, agg_v, den_v, disp_v):
    wid = lax.axis_index("s") * 2 + lax.axis_index("c")
    pltpu.sync_copy(idx_hbm.at[wid], idx_v)
    pltpu.sync_copy(vals_hbm.at[wid], vals_v)

    def group_body(j, _):
        m = wid * mpw + j
        pltpu.sync_copy(xv_hbm.at[m], xv_v)
        pltpu.sync_copy(cv_hbm.at[m], agg_v)
        for t in range(S // NLANE):
            den_v[pl.ds(t * NLANE, NLANE)] = jnp.full(
                (NLANE,), 1.0, jnp.float32)

        def scat_body(c, _):
            off = j * L + c * NLANE
            iv = idx_v[pl.ds(off, NLANE)]
            wv = vals_v[pl.ds(off, NLANE)]
            plsc.addupdate_scatter(den_v, [iv], wv)
            for col in range(SCH):
                xcol = xv_v[pl.ds(col * L + c * NLANE, NLANE)]
                plsc.addupdate_scatter(agg_v, [iv + col * S], xcol)
            return 0

        lax.fori_loop(0, L // NLANE, scat_body, 0)

        def disp_body(c, _):
            off = j * L + c * NLANE
            iv = idx_v[pl.ds(off, NLANE)]
            wv = vals_v[pl.ds(off, NLANE)]
            wr = wv / plsc.load_gather(den_v, [iv])
            for col in range(SCH):
                g = plsc.load_gather(agg_v, [iv + col * S])
                disp_v[pl.ds(col * L + c * NLANE, NLANE)] = wr * g
            return 0

        lax.fori_loop(0, L // NLANE, disp_body, 0)
        pltpu.sync_copy(disp_v, out_hbm.at[m])
        return 0

    lax.fori_loop(0, mpw, group_body, 0)


def _run_sim(a, W_proj, b_proj, ab, nf, interpret):
    return pl.pallas_call(
        _sim_kernel,
        grid=(nf,),
        in_specs=[
            pl.BlockSpec((1, L, C_IN), lambda f: (f, 0, 0)),
            pl.BlockSpec((C2, C_IN), lambda f: (0, 0)),
            pl.BlockSpec((C2,), lambda f: (0,)),
            pl.BlockSpec(memory_space=pltpu.SMEM),
        ],
        out_specs=[
            pl.BlockSpec((1, FC, L), lambda f: (f, 0, 0)),
            pl.BlockSpec((1, FC, L), lambda f: (f, 0, 0)),
            pl.BlockSpec((1, FC, SCH, L), lambda f: (f, 0, 0, 0)),
            pl.BlockSpec((1, FC, SCH, S), lambda f: (f, 0, 0, 0)),
        ],
        out_shape=[
            jax.ShapeDtypeStruct((nf, FC, L), jnp.float32),
            jax.ShapeDtypeStruct((nf, FC, L), jnp.int32),
            jax.ShapeDtypeStruct((nf, FC, SCH, L), jnp.float32),
            jax.ShapeDtypeStruct((nf, FC, SCH, S), jnp.float32),
        ],
        interpret=interpret,
    )(a, W_proj, b_proj, ab)


def _run_route(vals, idx, xv_t, cv_t, nf):
    m = nf * FC
    mpw = m // NW
    idx_w = idx.reshape(NW, mpw * L)
    vals_w = vals.reshape(NW, mpw * L)
    xv_w = xv_t.reshape(m, SCH * L)
    cv_w = cv_t.reshape(m, SCH * S)
    route = functools.partial(
        pl.kernel,
        mesh=plsc.VectorSubcoreMesh(core_axis_name="c", subcore_axis_name="s"),
        compiler_params=pltpu.CompilerParams(needs_layout_passes=False),
        out_type=jax.ShapeDtypeStruct((m, SCH * L), jnp.float32),
        scratch_types=[
            pltpu.VMEM((mpw * L,), jnp.int32),
            pltpu.VMEM((mpw * L,), jnp.float32),
            pltpu.VMEM((SCH * L,), jnp.float32),
            pltpu.VMEM((SCH * S,), jnp.float32),
            pltpu.VMEM((2 * S,), jnp.float32),
            pltpu.VMEM((SCH * L,), jnp.float32),
        ],
    )(functools.partial(_route_sc, mpw))
    return route(idx_w, vals_w, xv_w, cv_w)


def _run_merge(disp_cm, W_merge, b_merge, nf, interpret):
    return pl.pallas_call(
        _merge_kernel,
        grid=(nf,),
        in_specs=[
            pl.BlockSpec((1, FC * SCH, L), lambda f: (f, 0, 0)),
            pl.BlockSpec((C_IN, HD), lambda f: (0, 0)),
            pl.BlockSpec((C_IN,), lambda f: (0,)),
        ],
        out_specs=pl.BlockSpec((1, L, C_IN), lambda f: (f, 0, 0)),
        out_shape=jax.ShapeDtypeStruct((nf, L, C_IN), jnp.float32),
        interpret=interpret,
    )(disp_cm, W_merge, b_merge)


NSPLIT = 2


@functools.partial(jax.jit, static_argnames=("interpret",))
def kernel(x, W_proj, b_proj, W_merge, b_merge, alpha, beta, interpret=False):
    a = x.reshape(N, C_IN, FS, SH, FS, SW).transpose(0, 2, 4, 3, 5, 1)
    a = a.reshape(NFOLD, L, C_IN)
    ab = jnp.concatenate([alpha, beta]).astype(jnp.float32)
    nf = NFOLD // NSPLIT
    sims = [_run_sim(a[i * nf:(i + 1) * nf], W_proj, b_proj, ab, nf,
                     interpret)
            for i in range(NSPLIT)]
    disps = [_run_route(*sims[i], nf) for i in range(NSPLIT)]
    outs = [_run_merge(disps[i].reshape(nf, FC * SCH, L), W_merge, b_merge,
                       nf, interpret)
            for i in range(NSPLIT)]
    out = jnp.concatenate(outs, axis=0)
    out = out.reshape(N, FS, FS, SH, SW, C_IN).transpose(0, 5, 1, 3, 2, 4)
    return out.reshape(N, C_IN, H, W_)
